# baseline (device time: 17692 ns/iter reference)
import jax
import jax.numpy as jnp
from jax import lax
from jax.experimental import pallas as pl
from jax.experimental.pallas import tpu as pltpu

N_DEV = 8
N_CHUNK = 2


def kernel(A, B):
    m, k = A.shape
    _, n = B.shape
    seg = m // N_DEV
    nc = n // N_CHUNK

    def body(a_ref, b_ref, out_ref, pbf_ref,
             rbuf1, ssem1, rsem1, ssem2, rsem2):
        my = lax.axis_index("i")

        barrier_sem = pltpu.get_barrier_semaphore()
        for d in range(1, N_DEV):
            peer = lax.rem(my + d, N_DEV)
            pl.semaphore_signal(
                barrier_sem, inc=1,
                device_id=(peer,), device_id_type=pl.DeviceIdType.MESH,
            )
        pl.semaphore_wait(barrier_sem, N_DEV - 1)

        partial = jnp.dot(
            a_ref[...].astype(jnp.bfloat16),
            b_ref[...].astype(jnp.bfloat16),
            preferred_element_type=jnp.float32,
        )
        pbf_ref[...] = partial.astype(jnp.bfloat16)
        myseg = pbf_ref[pl.ds(my * seg, seg), :].astype(jnp.float32)

        rs = []
        for c in range(N_CHUNK):
            for d in range(1, N_DEV):
                peer = lax.rem(my + d, N_DEV)
                rdma = pltpu.make_async_remote_copy(
                    src_ref=pbf_ref.at[pl.ds(peer * seg, seg),
                                       pl.ds(c * nc, nc)],
                    dst_ref=rbuf1.at[c, d - 1],
                    send_sem=ssem1.at[c, d - 1],
                    recv_sem=rsem1.at[c, d - 1],
                    device_id=(peer,),
                    device_id_type=pl.DeviceIdType.MESH,
                )
                rdma.start()
                rs.append(rdma)

        ag = []
        for c in range(N_CHUNK):
            acc = myseg[:, c * nc:(c + 1) * nc]
            for d in range(1, N_DEV):
                rs[c * (N_DEV - 1) + d - 1].wait_recv()
                acc = acc + rbuf1[c, d - 1].astype(jnp.float32)
            acc = jnp.maximum(acc, 0.0)
            out_ref[pl.ds(my * seg, seg), pl.ds(c * nc, nc)] = (
                acc.astype(jnp.bfloat16)
            )
            for d in range(1, N_DEV):
                peer = lax.rem(my + d, N_DEV)
                rdma = pltpu.make_async_remote_copy(
                    src_ref=out_ref.at[pl.ds(my * seg, seg),
                                       pl.ds(c * nc, nc)],
                    dst_ref=out_ref.at[pl.ds(my * seg, seg),
                                       pl.ds(c * nc, nc)],
                    send_sem=ssem2.at[c, d - 1],
                    recv_sem=rsem2.at[c, d - 1],
                    device_id=(peer,),
                    device_id_type=pl.DeviceIdType.MESH,
                )
                rdma.start()
                ag.append(rdma)

        for r in ag:
            r.wait_recv()
        for r in rs:
            r.wait_send()
        for r in ag:
            r.wait_send()

    return pl.pallas_call(
        body,
        out_shape=jax.ShapeDtypeStruct((m, n), jnp.bfloat16),
        in_specs=[
            pl.BlockSpec(memory_space=pltpu.VMEM),
            pl.BlockSpec(memory_space=pltpu.VMEM),
        ],
        out_specs=pl.BlockSpec(memory_space=pltpu.VMEM),
        scratch_shapes=[
            pltpu.VMEM((m, n), jnp.bfloat16),
            pltpu.VMEM((N_CHUNK, N_DEV - 1, seg, nc), jnp.bfloat16),
            pltpu.SemaphoreType.DMA((N_CHUNK, N_DEV - 1)),
            pltpu.SemaphoreType.DMA((N_CHUNK, N_DEV - 1)),
            pltpu.SemaphoreType.DMA((N_CHUNK, N_DEV - 1)),
            pltpu.SemaphoreType.DMA((N_CHUNK, N_DEV - 1)),
        ],
        compiler_params=pltpu.CompilerParams(collective_id=0),
    )(A, B)
